# baseline (speedup 1.0000x reference)
import functools

import jax
import jax.numpy as jnp
from jax.experimental import pallas as pl
from jax.experimental.pallas import tpu as pltpu


def _issue_tile(ids_ref, table_hbm, emb_buf, sems, base, slot, tm):
    for r in range(tm):
        idx = ids_ref[base + r]
        pltpu.make_async_copy(
            table_hbm.at[pl.ds(idx, 1), :],
            emb_buf.at[slot, pl.ds(r, 1), :],
            sems.at[slot],
        ).start()


def _embed_project_body(ids_ref, table_hbm, w_ref, b_ref, out_ref,
                        emb_buf, sems, *, n_inner, tm):
    j = pl.program_id(0)

    @pl.when(j == 0)
    def _prime():
        _issue_tile(ids_ref, table_hbm, emb_buf, sems, j * tm, 0, tm)

    @pl.when(j + 1 < n_inner)
    def _prefetch():
        nxt = jax.lax.rem(j + 1, 2)
        _issue_tile(ids_ref, table_hbm, emb_buf, sems, (j + 1) * tm, nxt, tm)

    cur = jax.lax.rem(j, 2)
    pltpu.make_async_copy(
        table_hbm.at[pl.ds(0, tm), :], emb_buf.at[cur], sems.at[cur]
    ).wait()

    @pl.when(cur == 0)
    def _mm0():
        out_ref[...] = jnp.dot(emb_buf[0], w_ref[...],
                               preferred_element_type=jnp.float32) + b_ref[...]

    @pl.when(cur == 1)
    def _mm1():
        out_ref[...] = jnp.dot(emb_buf[1], w_ref[...],
                               preferred_element_type=jnp.float32) + b_ref[...]


@functools.partial(jax.jit, static_argnames=("tm",))
def _forward(ids, table, w, b, *, tm=256):
    B, S = ids.shape
    V, E = table.shape
    H = w.shape[1]
    N = B * S

    tm_eff = max(8, min(int(tm), ((N + 7) // 8) * 8))
    tm_eff = ((tm_eff + 7) // 8) * 8
    n_tiles = (N + tm_eff - 1) // tm_eff
    n_pad = n_tiles * tm_eff

    ids_flat = jnp.clip(ids.reshape(-1).astype(jnp.int32), 0, V - 1)
    if n_pad != N:
        ids_flat = jnp.pad(ids_flat, (0, n_pad - N))

    out = pl.pallas_call(
        functools.partial(_embed_project_body, n_inner=n_tiles, tm=tm_eff),
        out_shape=jax.ShapeDtypeStruct((n_pad, H), jnp.float32),
        grid_spec=pltpu.PrefetchScalarGridSpec(
            num_scalar_prefetch=1,
            grid=(n_tiles,),
            in_specs=[
                pl.BlockSpec(memory_space=pl.ANY),
                pl.BlockSpec((E, H), lambda j, ids: (0, 0)),
                pl.BlockSpec((1, H), lambda j, ids: (0, 0)),
            ],
            out_specs=pl.BlockSpec((tm_eff, H), lambda j, ids: (j, 0)),
            scratch_shapes=[
                pltpu.VMEM((2, tm_eff, E), table.dtype),
                pltpu.SemaphoreType.DMA((2,)),
            ],
        ),
        compiler_params=pltpu.CompilerParams(
            dimension_semantics=("arbitrary",),
            disable_bounds_checks=True,
        ),
    )(ids_flat, table, w, b)
    return out[:N].reshape(B, S, H)


def kernel(ids, table, w, b):
    return _forward(ids, table, w, b, tm=4096)


# speedup vs baseline: 1.0461x; 1.0461x over previous
import functools

import jax
import jax.numpy as jnp
from jax.experimental import pallas as pl
from jax.experimental.pallas import tpu as pltpu


def _issue_tile(ids_ref, table_hbm, emb_buf, sems, base, slot, tm):
    for r in range(tm):
        idx = ids_ref[base + r]
        pltpu.make_async_copy(
            table_hbm.at[pl.ds(idx, 1), :],
            emb_buf.at[slot, pl.ds(r, 1), :],
            sems.at[slot],
        ).start()


def _embed_project_body(ids_ref, table_hbm, w_ref, b_ref, out_ref,
                        emb_buf, sems, *, n_inner, tm):
    j = pl.program_id(0)

    @pl.when(j == 0)
    def _prime():
        _issue_tile(ids_ref, table_hbm, emb_buf, sems, j * tm, 0, tm)

    @pl.when(j + 1 < n_inner)
    def _prefetch():
        nxt = jax.lax.rem(j + 1, 2)
        _issue_tile(ids_ref, table_hbm, emb_buf, sems, (j + 1) * tm, nxt, tm)

    cur = jax.lax.rem(j, 2)
    pltpu.make_async_copy(
        table_hbm.at[pl.ds(0, tm), :], emb_buf.at[cur], sems.at[cur]
    ).wait()

    @pl.when(cur == 0)
    def _mm0():
        out_ref[...] = jnp.dot(emb_buf[0], w_ref[...],
                               preferred_element_type=jnp.float32) + b_ref[...]

    @pl.when(cur == 1)
    def _mm1():
        out_ref[...] = jnp.dot(emb_buf[1], w_ref[...],
                               preferred_element_type=jnp.float32) + b_ref[...]


@functools.partial(jax.jit, static_argnames=("tm",))
def _forward(ids, table, w, b, *, tm=256):
    B, S = ids.shape
    V, E = table.shape
    H = w.shape[1]
    N = B * S

    tm_eff = max(8, min(int(tm), ((N + 7) // 8) * 8))
    tm_eff = ((tm_eff + 7) // 8) * 8
    n_tiles = (N + tm_eff - 1) // tm_eff
    n_pad = n_tiles * tm_eff

    ids_flat = jnp.clip(ids.reshape(-1).astype(jnp.int32), 0, V - 1)
    if n_pad != N:
        ids_flat = jnp.pad(ids_flat, (0, n_pad - N))

    out = pl.pallas_call(
        functools.partial(_embed_project_body, n_inner=n_tiles, tm=tm_eff),
        out_shape=jax.ShapeDtypeStruct((n_pad, H), jnp.float32),
        grid_spec=pltpu.PrefetchScalarGridSpec(
            num_scalar_prefetch=1,
            grid=(n_tiles,),
            in_specs=[
                pl.BlockSpec(memory_space=pl.ANY),
                pl.BlockSpec((E, H), lambda j, ids: (0, 0)),
                pl.BlockSpec((1, H), lambda j, ids: (0, 0)),
            ],
            out_specs=pl.BlockSpec((tm_eff, H), lambda j, ids: (j, 0)),
            scratch_shapes=[
                pltpu.VMEM((2, tm_eff, E), table.dtype),
                pltpu.SemaphoreType.DMA((2,)),
            ],
        ),
        compiler_params=pltpu.CompilerParams(
            dimension_semantics=("arbitrary",),
            disable_bounds_checks=True,
        ),
    )(ids_flat, table, w, b)
    return out[:N].reshape(B, S, H)


def kernel(ids, table, w, b):
    return _forward(ids, table, w, b, tm=2048)
